# baseline (speedup 1.0000x reference)
import jax
import jax.numpy as jnp
from jax.experimental import pallas as pl
from jax.experimental.pallas import tpu as pltpu

N = 100000
D_IN = 128
D_HID = 256
D_OUT = 128

BLOCKS = [10000, 10000, 10000, 10000, 10000, 10000, 8000, 6400, 5600, 4800,
          4000, 3200, 2400, 2000, 1600, 1200, 800]
assert sum(BLOCKS) == N
MAXB = max(BLOCKS)
OFFS = [sum(BLOCKS[:k]) for k in range(len(BLOCKS))]
K_IN = 4
K_OUT = 3


def _outer(x_hbm, w1_ref, b1_ref, w2_ref, b2_ref, out_hbm, ibuf, obuf, isem, osem):
    w1b = w1_ref[...].astype(jnp.bfloat16)
    w2b = w2_ref[...].astype(jnp.bfloat16)
    b1v = b1_ref[...]
    b2v = b2_ref[...]

    def _in_copy(step):
        return pltpu.make_async_copy(
            x_hbm.at[pl.ds(OFFS[step], BLOCKS[step]), :],
            ibuf.at[step % K_IN, pl.ds(0, BLOCKS[step]), :],
            isem.at[step % K_IN],
        )

    def _out_copy(step):
        return pltpu.make_async_copy(
            obuf.at[step % K_OUT, pl.ds(0, BLOCKS[step]), :],
            out_hbm.at[pl.ds(OFFS[step], BLOCKS[step]), :],
            osem.at[step % K_OUT],
        )

    for j in range(K_IN):
        _in_copy(j).start()

    for i in range(len(BLOCKS)):
        if i >= K_OUT:
            _out_copy(i - K_OUT).wait()
        _in_copy(i).wait()
        xb = ibuf[i % K_IN, pl.ds(0, BLOCKS[i]), :].astype(jnp.bfloat16)
        h = jnp.dot(xb, w1b, preferred_element_type=jnp.float32)
        h = jnp.maximum(h + b1v, 0.0).astype(jnp.bfloat16)
        out = jnp.dot(h, w2b, preferred_element_type=jnp.float32)
        obuf[i % K_OUT, pl.ds(0, BLOCKS[i]), :] = out + b2v
        _out_copy(i).start()
        if i + K_IN < len(BLOCKS):
            _in_copy(i + K_IN).start()

    for j in range(len(BLOCKS) - K_OUT, len(BLOCKS)):
        _out_copy(j).wait()


@jax.jit
def kernel(x, W1, b1, W2, b2):
    b1r = b1.reshape(1, D_HID)
    b2r = b2.reshape(1, D_OUT)
    return pl.pallas_call(
        _outer,
        in_specs=[
            pl.BlockSpec(memory_space=pltpu.MemorySpace.HBM),
            pl.BlockSpec(memory_space=pltpu.MemorySpace.VMEM),
            pl.BlockSpec(memory_space=pltpu.MemorySpace.VMEM),
            pl.BlockSpec(memory_space=pltpu.MemorySpace.VMEM),
            pl.BlockSpec(memory_space=pltpu.MemorySpace.VMEM),
        ],
        out_specs=pl.BlockSpec(memory_space=pltpu.MemorySpace.HBM),
        out_shape=jax.ShapeDtypeStruct((N, D_OUT), jnp.float32),
        scratch_shapes=[
            pltpu.VMEM((K_IN, MAXB, D_IN), jnp.float32),
            pltpu.VMEM((K_OUT, MAXB, D_OUT), jnp.float32),
            pltpu.SemaphoreType.DMA((K_IN,)),
            pltpu.SemaphoreType.DMA((K_OUT,)),
        ],
    )(x, W1, b1r, W2, b2r)


# speedup vs baseline: 1.0494x; 1.0494x over previous
import jax
import jax.numpy as jnp
from jax.experimental import pallas as pl
from jax.experimental.pallas import tpu as pltpu

N = 100000
D_IN = 128
D_HID = 256
D_OUT = 128
BLOCK = 5000
NBUF = 6
K_OUT = 4
STEPS = N // (2 * BLOCK)


def _outer(x_hbm, w1_ref, b1_ref, w2_ref, b2_ref, out_hbm, obuf_a, obuf_b, osem):
    w1b = w1_ref[...].astype(jnp.bfloat16)
    w2b = w2_ref[...].astype(jnp.bfloat16)
    b1v = b1_ref[...]
    b2v = b2_ref[...]

    def _copy_a(step, slot):
        return pltpu.make_async_copy(
            obuf_a.at[slot],
            out_hbm.at[pl.ds(step * BLOCK, BLOCK), :],
            osem.at[0, slot],
        )

    def _copy_b(step, slot):
        return pltpu.make_async_copy(
            obuf_b.at[slot],
            out_hbm.at[pl.ds((STEPS + step) * BLOCK, BLOCK), :],
            osem.at[1, slot],
        )

    def _mlp(x_f32):
        xb = x_f32.astype(jnp.bfloat16)
        h = jnp.dot(xb, w1b, preferred_element_type=jnp.float32)
        h = jnp.maximum(h + b1v, 0.0).astype(jnp.bfloat16)
        out = jnp.dot(h, w2b, preferred_element_type=jnp.float32)
        return out + b2v

    def inner(idxs, xa_ref, xb_ref):
        i = idxs[0]
        slot = jax.lax.rem(i, K_OUT)

        @pl.when(i >= K_OUT)
        def _wait_prev():
            _copy_a(i - K_OUT, slot).wait()
            _copy_b(i - K_OUT, slot).wait()

        obuf_a[slot] = _mlp(xa_ref[...])
        _copy_a(i, slot).start()
        obuf_b[slot] = _mlp(xb_ref[...])
        _copy_b(i, slot).start()

    pltpu.emit_pipeline(
        inner,
        grid=(STEPS,),
        in_specs=[
            pl.BlockSpec(
                (BLOCK, D_IN), lambda i: (i, 0),
                pipeline_mode=pl.Buffered(buffer_count=NBUF),
            ),
            pl.BlockSpec(
                (BLOCK, D_IN), lambda i: (STEPS + i, 0),
                pipeline_mode=pl.Buffered(buffer_count=NBUF),
            ),
        ],
        out_specs=[],
        _explicit_indices=True,
    )(x_hbm, x_hbm)

    for j in range(max(0, STEPS - K_OUT), STEPS):
        _copy_a(j, j % K_OUT).wait()
        _copy_b(j, j % K_OUT).wait()


@jax.jit
def kernel(x, W1, b1, W2, b2):
    b1r = b1.reshape(1, D_HID)
    b2r = b2.reshape(1, D_OUT)
    return pl.pallas_call(
        _outer,
        in_specs=[
            pl.BlockSpec(memory_space=pltpu.MemorySpace.HBM),
            pl.BlockSpec(memory_space=pltpu.MemorySpace.VMEM),
            pl.BlockSpec(memory_space=pltpu.MemorySpace.VMEM),
            pl.BlockSpec(memory_space=pltpu.MemorySpace.VMEM),
            pl.BlockSpec(memory_space=pltpu.MemorySpace.VMEM),
        ],
        out_specs=pl.BlockSpec(memory_space=pltpu.MemorySpace.HBM),
        out_shape=jax.ShapeDtypeStruct((N, D_OUT), jnp.float32),
        scratch_shapes=[
            pltpu.VMEM((K_OUT, BLOCK, D_OUT), jnp.float32),
            pltpu.VMEM((K_OUT, BLOCK, D_OUT), jnp.float32),
            pltpu.SemaphoreType.DMA((2, K_OUT)),
        ],
    )(x, W1, b1r, W2, b2r)


# speedup vs baseline: 1.0522x; 1.0026x over previous
import jax
import jax.numpy as jnp
from jax.experimental import pallas as pl
from jax.experimental.pallas import tpu as pltpu

N = 100000
D_IN = 128
D_HID = 256
D_OUT = 128
BLOCK = 5000
NBUF = 6
K_OUT = 4
STEPS = N // (2 * BLOCK)


def _outer(x_hbm, w1_ref, b1_ref, w2_ref, b2_ref, out_hbm, obuf_a, obuf_b, osem):
    w1b = w1_ref[...].astype(jnp.bfloat16)
    w2b = w2_ref[...].astype(jnp.bfloat16)
    b1v = b1_ref[...]
    b2v = b2_ref[...]

    def _copy_a(step, slot):
        return pltpu.make_async_copy(
            obuf_a.at[slot],
            out_hbm.at[pl.ds(step * BLOCK, BLOCK), :],
            osem.at[0, slot],
        )

    def _copy_b(step, slot):
        return pltpu.make_async_copy(
            obuf_b.at[slot],
            out_hbm.at[pl.ds((STEPS + step) * BLOCK, BLOCK), :],
            osem.at[1, slot],
        )

    def _mlp(x_f32):
        h = jnp.dot(x_f32, w1_ref[...], preferred_element_type=jnp.float32)
        h = jnp.maximum(h + b1v, 0.0).astype(jnp.bfloat16)
        out = jnp.dot(h, w2b, preferred_element_type=jnp.float32)
        return out + b2v

    def inner(idxs, xa_ref, xb_ref):
        i = idxs[0]
        slot = jax.lax.rem(i, K_OUT)

        @pl.when(i >= K_OUT)
        def _wait_prev():
            _copy_a(i - K_OUT, slot).wait()
            _copy_b(i - K_OUT, slot).wait()

        obuf_a[slot] = _mlp(xa_ref[...])
        _copy_a(i, slot).start()
        obuf_b[slot] = _mlp(xb_ref[...])
        _copy_b(i, slot).start()

    pltpu.emit_pipeline(
        inner,
        grid=(STEPS,),
        in_specs=[
            pl.BlockSpec(
                (BLOCK, D_IN), lambda i: (i, 0),
                pipeline_mode=pl.Buffered(buffer_count=NBUF),
            ),
            pl.BlockSpec(
                (BLOCK, D_IN), lambda i: (STEPS + i, 0),
                pipeline_mode=pl.Buffered(buffer_count=NBUF),
            ),
        ],
        out_specs=[],
        _explicit_indices=True,
    )(x_hbm, x_hbm)

    for j in range(max(0, STEPS - K_OUT), STEPS):
        _copy_a(j, j % K_OUT).wait()
        _copy_b(j, j % K_OUT).wait()


@jax.jit
def kernel(x, W1, b1, W2, b2):
    b1r = b1.reshape(1, D_HID)
    b2r = b2.reshape(1, D_OUT)
    return pl.pallas_call(
        _outer,
        in_specs=[
            pl.BlockSpec(memory_space=pltpu.MemorySpace.HBM),
            pl.BlockSpec(memory_space=pltpu.MemorySpace.VMEM),
            pl.BlockSpec(memory_space=pltpu.MemorySpace.VMEM),
            pl.BlockSpec(memory_space=pltpu.MemorySpace.VMEM),
            pl.BlockSpec(memory_space=pltpu.MemorySpace.VMEM),
        ],
        out_specs=pl.BlockSpec(memory_space=pltpu.MemorySpace.HBM),
        out_shape=jax.ShapeDtypeStruct((N, D_OUT), jnp.float32),
        scratch_shapes=[
            pltpu.VMEM((K_OUT, BLOCK, D_OUT), jnp.float32),
            pltpu.VMEM((K_OUT, BLOCK, D_OUT), jnp.float32),
            pltpu.SemaphoreType.DMA((2, K_OUT)),
        ],
    )(x, W1, b1r, W2, b2r)
